# baseline (speedup 1.0000x reference)
import functools

import jax
import jax.numpy as jnp
from jax import lax
from jax.experimental import pallas as pl
from jax.experimental.pallas import tpu as pltpu
from jax.experimental.pallas import tpu_sc as plsc

E = 8
TEMP = 0.1
ALPHA = 0.3
UNIF = ALPHA / E
KNOWN_V = 1.0 - ALPHA + UNIF
TTYPE_UNKNOWN = 5
INV7 = 1.0 / 7.0

NC, NS, L = 2, 16, 16
NW = NC * NS
CH = 64
MBLK = 128
MAXB = 1024 // MBLK


def _wid():
    return lax.axis_index("s") * NC + lax.axis_index("c")


def _sca_body(types_hbm, tokens_hbm, counts_hbm, xc_hbm,
              types_v, idx_v, idxc_v, rows_v, cnt_v, sem, *, tpw):
    w = _wid()
    base = w * tpw
    pltpu.sync_copy(types_hbm.at[pl.ds(base, tpw)], types_v)

    zz = jnp.zeros((L,), jnp.int32)

    def zinit(g, c):
        idx_v[pl.ds(g * L, L)] = zz
        return c
    lax.fori_loop(0, (tpw + L) // L, zinit, 0)

    il = lax.iota(jnp.int32, L)

    def comp(g, cnt):
        tv = types_v[pl.ds(g * L, L)]
        m = tv == TTYPE_UNKNOWN
        mi = m.astype(jnp.int32)
        excl = plsc.cumsum(mi) - mi
        gi = base + g * L + il
        plsc.store_scatter(idx_v, [cnt + excl], gi, mask=m)
        return cnt + jnp.sum(mi)
    cnt = lax.fori_loop(0, tpw // L, comp, jnp.int32(0))

    cnt_v[...] = jnp.full((L,), cnt, jnp.int32)
    pltpu.sync_copy(cnt_v, counts_hbm.at[w])

    def gat(c, carry):
        def cp(k, c2):
            idxc_v[pl.ds(k * L, L)] = idx_v[pl.ds(c * CH + k * L, L)]
            return c2
        lax.fori_loop(0, CH // L, cp, 0)
        pltpu.async_copy(tokens_hbm.at[idxc_v], rows_v, sem).wait()
        pltpu.sync_copy(rows_v, xc_hbm.at[pl.ds(base + c * CH, CH)])
        return carry
    lax.fori_loop(0, (cnt + CH - 1) // CH, gat, 0)


def _gelu_exact(x):
    return 0.5 * x * (1.0 + jax.lax.erf(x * 0.7071067811865476))


def _tc_body(counts_ref, w1_ref, b1_ref, w2_ref, b2_ref, xc_ref,
             jv_ref, wl_ref, bufs, sems, *, tpw):
    def outer(w, k):
        cnt = counts_ref[w, 0]
        nb = (cnt + MBLK - 1) // MBLK

        def inner(c, k2):
            wl_ref[k2] = w * tpw + c * MBLK
            return k2 + 1
        return lax.fori_loop(0, nb, inner, k)
    nblk = lax.fori_loop(0, NW, outer, jnp.int32(0))

    def issue(k, slot):
        rs = pl.multiple_of(wl_ref[k], MBLK)
        pltpu.make_async_copy(xc_ref.at[pl.ds(rs, MBLK)],
                              bufs.at[slot], sems.at[slot]).start()

    @pl.when(nblk > 0)
    def _():
        issue(0, 0)

    col = lax.broadcasted_iota(jnp.int32, (MBLK, E), 1)

    def compute(k, slot):
        rs = pl.multiple_of(wl_ref[k], MBLK)
        pltpu.make_async_copy(xc_ref.at[pl.ds(rs, MBLK)],
                              bufs.at[slot], sems.at[slot]).wait()

        @pl.when(k + 1 < nblk)
        def _():
            issue(k + 1, 1 - slot)

        x = bufs[slot]
        h = jnp.dot(x, w1_ref[...], preferred_element_type=jnp.float32)
        h = _gelu_exact(h + b1_ref[...])
        logits = (jnp.dot(h, w2_ref[...], preferred_element_type=jnp.float32)
                  + b2_ref[...]) * TEMP
        m = jnp.max(logits, axis=-1, keepdims=True)
        sumexp = jnp.sum(jnp.exp(logits - m), axis=-1, keepdims=True)
        j = jnp.min(jnp.where(logits >= m, col, E), axis=-1, keepdims=True)
        v = (1.0 - ALPHA) / sumexp + UNIF
        out = jnp.where(col == 0, v,
                        jnp.where(col == 1, j.astype(jnp.float32), 0.0))
        jv_ref[pl.ds(rs, MBLK), :] = out

    def step2(p, carry):
        for s in range(2):
            k = 2 * p + s

            @pl.when(k < nblk)
            def _():
                compute(k, s)
        return carry
    lax.fori_loop(0, (nblk + 1) // 2, step2, 0)


def _scb_body(types_hbm, jv_hbm, t_hbm, disp_hbm, comb_hbm,
              types_v, jv_v, t_v, disp_v, comb_v, *, tpw, wpb):
    w = _wid()
    base = w * tpw
    pltpu.sync_copy(types_hbm.at[pl.ds(base, tpw)], types_v)
    pltpu.sync_copy(jv_hbm.at[pl.ds(base * E, tpw * E)], jv_v)
    pltpu.sync_copy(t_hbm, t_v)

    b = w // wpb
    il = lax.iota(jnp.int32, L)
    tf = t_v[...].astype(jnp.float32)
    cap = jnp.sum(jnp.where(il == b, 0.5 + 1.0e-4 * tf, 0.0))

    def grp(g, cnt):
        tv = types_v[pl.ds(g * L, L)]
        m = tv == TTYPE_UNKNOWN
        mi = m.astype(jnp.int32)
        excl = plsc.cumsum(mi) - mi
        r = jnp.minimum(cnt + excl, tpw - 1)
        vs = plsc.load_gather(jv_v, [r * E], mask=m)
        js = plsc.load_gather(jv_v, [r * E + 1], mask=m)
        v = jnp.where(m, vs, KNOWN_V)
        j = jnp.where(m, js.astype(jnp.int32), tv)
        hi = jnp.minimum(v, cap)
        lo = jnp.maximum(v - cap, 0.0) * INV7
        s = 1.0 / (hi + 7.0 * lo + 1e-8)
        flat = (g * L + il) * E
        for e in range(E):
            de = jnp.where(j == e, hi, lo)
            plsc.store_scatter(disp_v, [flat + e], de)
            plsc.store_scatter(comb_v, [flat + e], de * s)
        return cnt + jnp.sum(mi)
    lax.fori_loop(0, tpw // L, grp, jnp.int32(0))

    pltpu.sync_copy(disp_v, disp_hbm.at[pl.ds(base * E, tpw * E)])
    pltpu.sync_copy(comb_v, comb_hbm.at[pl.ds(base * E, tpw * E)])


def _sc_mesh():
    return plsc.VectorSubcoreMesh(core_axis_name="c", subcore_axis_name="s",
                                  num_cores=NC, num_subcores=NS)


def _run_sca(types, x):
    T, D = x.shape
    TPW = T // NW
    sca = pl.kernel(
        functools.partial(_sca_body, tpw=TPW),
        out_type=[
            jax.ShapeDtypeStruct((NW, L), jnp.int32),
            jax.ShapeDtypeStruct((T, D), jnp.float32),
        ],
        mesh=_sc_mesh(),
        scratch_types=[
            pltpu.VMEM((TPW,), jnp.int32),
            pltpu.VMEM((TPW + L,), jnp.int32),
            pltpu.VMEM((CH,), jnp.int32),
            pltpu.VMEM((CH, D), jnp.float32),
            pltpu.VMEM((L,), jnp.int32),
            pltpu.SemaphoreType.DMA,
        ],
        compiler_params=pltpu.CompilerParams(needs_layout_passes=False),
    )
    return sca(types, x)


def _run_tc(counts, W1, b1, W2, b2, xc):
    T, D = xc.shape
    TPW = T // NW
    return pl.pallas_call(
        functools.partial(_tc_body, tpw=TPW),
        grid=(),
        in_specs=[
            pl.BlockSpec(memory_space=pltpu.SMEM),
            pl.BlockSpec(memory_space=pltpu.VMEM),
            pl.BlockSpec(memory_space=pltpu.VMEM),
            pl.BlockSpec(memory_space=pltpu.VMEM),
            pl.BlockSpec(memory_space=pltpu.VMEM),
            pl.BlockSpec(memory_space=pl.ANY),
        ],
        out_specs=pl.BlockSpec(memory_space=pltpu.VMEM),
        out_shape=jax.ShapeDtypeStruct((T, E), jnp.float32),
        scratch_shapes=[
            pltpu.SMEM((NW * MAXB,), jnp.int32),
            pltpu.VMEM((2, MBLK, D), jnp.float32),
            pltpu.SemaphoreType.DMA((2,)),
        ],
    )(counts, W1, b1, W2, b2, xc)


def _run_scb(types, jv_flat, t16, nbatch):
    T = types.shape[0]
    TPW = T // NW
    WPB = NW // nbatch
    scb = pl.kernel(
        functools.partial(_scb_body, tpw=TPW, wpb=WPB),
        out_type=[
            jax.ShapeDtypeStruct((T * E,), jnp.float32),
            jax.ShapeDtypeStruct((T * E,), jnp.float32),
        ],
        mesh=_sc_mesh(),
        scratch_types=[
            pltpu.VMEM((TPW,), jnp.int32),
            pltpu.VMEM((TPW * E,), jnp.float32),
            pltpu.VMEM((L,), jnp.int32),
            pltpu.VMEM((TPW * E,), jnp.float32),
            pltpu.VMEM((TPW * E,), jnp.float32),
        ],
        compiler_params=pltpu.CompilerParams(needs_layout_passes=False),
    )
    return scb(types, jv_flat, t16)


def kernel(tokens, token_types, t, W1, b1, W2, b2):
    B, N, D = tokens.shape
    T = B * N

    x = tokens.reshape(T, D)
    types = token_types.reshape(T).astype(jnp.int32)
    t16 = jnp.zeros((L,), jnp.int32).at[:B].set(t.astype(jnp.int32))

    counts, xc = _run_sca(types, x)
    jv = _run_tc(counts, W1, b1, W2, b2, xc)
    disp, comb = _run_scb(types, jv.reshape(T * E), t16, B)

    return disp.reshape(B, N, E), comb.reshape(B, N, E)


# speedup vs baseline: 1.4009x; 1.4009x over previous
import functools

import jax
import jax.numpy as jnp
from jax import lax
from jax.experimental import pallas as pl
from jax.experimental.pallas import tpu as pltpu
from jax.experimental.pallas import tpu_sc as plsc

E = 8
TEMP = 0.1
ALPHA = 0.3
UNIF = ALPHA / E
KNOWN_V = 1.0 - ALPHA + UNIF
TTYPE_UNKNOWN = 5
INV7 = 1.0 / 7.0

NC, NS, L = 2, 16, 16
NW = NC * NS
CH = 64
MBLK = 128
MAXB = 1024 // MBLK


def _wid():
    return lax.axis_index("s") * NC + lax.axis_index("c")


def _sca_body(types_hbm, tokens_hbm, counts_hbm, xc_hbm,
              types_v, idx_v, idxc0, idxc1, rows0, rows1, cnt_v, gsem,
              *, tpw):
    w = _wid()
    base = w * tpw
    pltpu.sync_copy(types_hbm.at[pl.ds(base, tpw)], types_v)

    zz = jnp.zeros((L,), jnp.int32)

    def zinit(g, c):
        idx_v[pl.ds(g * L, L)] = zz
        return c
    lax.fori_loop(0, (tpw + L) // L, zinit, 0)

    il = lax.iota(jnp.int32, L)

    def comp(g, cnt):
        tv = types_v[pl.ds(g * L, L)]
        m = tv == TTYPE_UNKNOWN
        mi = m.astype(jnp.int32)
        excl = plsc.cumsum(mi) - mi
        gi = base + g * L + il
        plsc.store_scatter(idx_v, [cnt + excl], gi, mask=m)
        return cnt + jnp.sum(mi)
    cnt = lax.fori_loop(0, tpw // L, comp, jnp.int32(0))

    cnt_v[...] = jnp.full((L,), cnt, jnp.int32)
    pltpu.sync_copy(cnt_v, counts_hbm.at[w])

    nch = (cnt + CH - 1) // CH

    def fire(c, ic, rv, sm):
        def cp(k, c2):
            ic[pl.ds(k * L, L)] = idx_v[pl.ds(c * CH + k * L, L)]
            return c2
        lax.fori_loop(0, CH // L, cp, 0)
        pltpu.make_async_copy(tokens_hbm.at[ic], rv, sm).start()

    def drain(c, ic, rv, sm):
        pltpu.make_async_copy(tokens_hbm.at[ic], rv, sm).wait()
        pltpu.sync_copy(rv, xc_hbm.at[pl.ds(base + c * CH, CH)])

    slots = ((idxc0, rows0, gsem.at[0]), (idxc1, rows1, gsem.at[1]))

    @pl.when(nch > 0)
    def _():
        fire(0, *slots[0])

    def gat(p, carry):
        for s in range(2):
            c = 2 * p + s

            @pl.when(c < nch)
            def _():
                @pl.when(c + 1 < nch)
                def _():
                    fire(c + 1, *slots[1 - s])
                drain(c, *slots[s])
        return carry
    lax.fori_loop(0, (nch + 1) // 2, gat, 0)


_ERF_P = (7.85386146e-05, -0.000801019371, 0.00518832775, -0.0268538129,
          0.112835854, -0.37612626, 1.12837911)
_ERFC_Q1 = (0.0232682, -0.138703942, 0.368742466, -0.582473278, 0.621000469,
            -0.494451523, 0.340488, -0.274112701, 0.563825965)
_ERFC_Q2 = (-10.477664, 12.9772, -7.49551868, 2.92101908, -1.01526523,
            0.42184633, -0.282076746, 0.564189494)


def _erfc(x):
    ax = jnp.abs(x)
    x2 = x * x
    p = x2 * _ERF_P[0]
    for c in _ERF_P[1:-1]:
        p = (p + c) * x2
    p = p + _ERF_P[-1]
    small = 1.0 - x * p

    nx2 = -x2
    zr = jnp.exp(nx2) * (1.0 / ax)
    u = 1.0 / x2
    q1 = u * _ERFC_Q1[0]
    for c in _ERFC_Q1[1:-1]:
        q1 = (q1 + c) * u
    q1 = q1 + _ERFC_Q1[-1]
    q2 = u * _ERFC_Q2[0]
    for c in _ERFC_Q2[1:-1]:
        q2 = (q2 + c) * u
    q2 = q2 + _ERFC_Q2[-1]
    lg = zr * jnp.where(ax < 2.0, q1, q2)
    lg = jnp.where(nx2 < -88.7228394, 0.0, lg)
    lg = jnp.where(x < 0.0, 2.0 - lg, lg)
    return jnp.where(ax < 1.0, small, lg)


def _gelu_exact(x):
    return 0.5 * x * _erfc(-x * 0.7071067811865476)


def _tc_body(counts_ref, w1_ref, b1_ref, w2_ref, b2_ref, xc_ref,
             jv_ref, wl_ref, bufs, parts, sems, *, tpw):
    def outer(w, k):
        cnt = counts_ref[w, 0]
        nb = (cnt + MBLK - 1) // MBLK

        def inner(c, k2):
            wl_ref[k2] = w * tpw + c * MBLK
            return k2 + 1
        return lax.fori_loop(0, nb, inner, k)
    nblk = lax.fori_loop(0, NW, outer, jnp.int32(0))

    def issue(k, slot):
        rs = pl.multiple_of(wl_ref[k], MBLK)
        pltpu.make_async_copy(xc_ref.at[pl.ds(rs, MBLK)],
                              bufs.at[slot], sems.at[slot]).start()

    @pl.when(nblk > 0)
    def _():
        issue(0, 0)

    row = lax.broadcasted_iota(jnp.int32, (E, MBLK), 0)

    def compute(k, slot):
        rs = pl.multiple_of(wl_ref[k], MBLK)
        pltpu.make_async_copy(xc_ref.at[pl.ds(rs, MBLK)],
                              bufs.at[slot], sems.at[slot]).wait()

        @pl.when(k + 1 < nblk)
        def _():
            issue(k + 1, 1 - slot)

        x = bufs[slot]
        w1 = w1_ref[...]
        for i in range(3):
            parts[i] = jnp.dot(x[:, i * 256:(i + 1) * 256],
                               w1[i * 256:(i + 1) * 256, :],
                               preferred_element_type=jnp.float32)
        h = (parts[0] + parts[1]) + parts[2]
        h = _gelu_exact(h + b1_ref[...])
        logits = (jnp.dot(h, w2_ref[...], preferred_element_type=jnp.float32)
                  + b2_ref[...]) * TEMP
        lt = logits.T
        m = jnp.max(lt, axis=0, keepdims=True)
        e = jnp.exp(lt - m)
        s = jnp.sum(e, axis=0, keepdims=True)
        pf = (1.0 - ALPHA) * (e / s) + UNIF
        mv = jnp.max(pf, axis=0, keepdims=True)
        j = jnp.min(jnp.where(pf >= mv, row, E), axis=0, keepdims=True)
        out = jnp.concatenate([mv, j.astype(jnp.float32)], axis=0)
        jv_ref[:, pl.ds(rs, MBLK)] = out

    def step2(p, carry):
        for s in range(2):
            k = 2 * p + s

            @pl.when(k < nblk)
            def _():
                compute(k, s)
        return carry
    lax.fori_loop(0, (nblk + 1) // 2, step2, 0)


def _scb_body(types_hbm, jv_hbm, t_hbm, disp_hbm, comb_hbm,
              types_v, v_v, j_v, t_v, disp_v, comb_v, *, tpw, wpb, n):
    w = _wid()
    base = w * tpw
    b = w // wpb
    tib = base - b * n
    pltpu.sync_copy(types_hbm.at[pl.ds(base, tpw)], types_v)
    pltpu.sync_copy(jv_hbm.at[0, pl.ds(base, tpw)], v_v)
    pltpu.sync_copy(jv_hbm.at[1, pl.ds(base, tpw)], j_v)
    pltpu.sync_copy(t_hbm, t_v)

    il = lax.iota(jnp.int32, L)
    tf = t_v[...].astype(jnp.float32)
    cap = jnp.sum(jnp.where(il == b, 0.5 + 1.0e-4 * tf, 0.0))

    def grp(g, cnt):
        tv = types_v[pl.ds(g * L, L)]
        m = tv == TTYPE_UNKNOWN
        mi = m.astype(jnp.int32)
        excl = plsc.cumsum(mi) - mi
        r = jnp.minimum(cnt + excl, tpw - 1)
        vs = plsc.load_gather(v_v, [r], mask=m)
        js = plsc.load_gather(j_v, [r], mask=m)
        v = jnp.where(m, vs, KNOWN_V)
        j = jnp.where(m, js.astype(jnp.int32), tv)
        hi = jnp.minimum(v, cap)
        lo = jnp.maximum(v - cap, 0.0) * INV7
        s = 1.0 / (hi + 7.0 * lo + 1e-8)
        for e in range(E):
            de = jnp.where(j == e, hi, lo)
            disp_v[e, pl.ds(g * L, L)] = de
            comb_v[e, pl.ds(g * L, L)] = de * s
        return cnt + jnp.sum(mi)
    lax.fori_loop(0, tpw // L, grp, jnp.int32(0))

    pltpu.sync_copy(disp_v, disp_hbm.at[pl.ds(b * E, E), pl.ds(tib, tpw)])
    pltpu.sync_copy(comb_v, comb_hbm.at[pl.ds(b * E, E), pl.ds(tib, tpw)])


def _sc_mesh():
    return plsc.VectorSubcoreMesh(core_axis_name="c", subcore_axis_name="s",
                                  num_cores=NC, num_subcores=NS)


def _run_sca(types, x):
    T, D = x.shape
    TPW = T // NW
    sca = pl.kernel(
        functools.partial(_sca_body, tpw=TPW),
        out_type=[
            jax.ShapeDtypeStruct((NW, L), jnp.int32),
            jax.ShapeDtypeStruct((T, D), jnp.float32),
        ],
        mesh=_sc_mesh(),
        scratch_types=[
            pltpu.VMEM((TPW,), jnp.int32),
            pltpu.VMEM((TPW + L,), jnp.int32),
            pltpu.VMEM((CH,), jnp.int32),
            pltpu.VMEM((CH,), jnp.int32),
            pltpu.VMEM((CH, D), jnp.float32),
            pltpu.VMEM((CH, D), jnp.float32),
            pltpu.VMEM((L,), jnp.int32),
            pltpu.SemaphoreType.DMA((2,)),
        ],
        compiler_params=pltpu.CompilerParams(needs_layout_passes=False),
    )
    return sca(types, x)


def _run_tc(counts, W1, b1, W2, b2, xc):
    T, D = xc.shape
    TPW = T // NW
    return pl.pallas_call(
        functools.partial(_tc_body, tpw=TPW),
        grid=(),
        in_specs=[
            pl.BlockSpec(memory_space=pltpu.SMEM),
            pl.BlockSpec(memory_space=pltpu.VMEM),
            pl.BlockSpec(memory_space=pltpu.VMEM),
            pl.BlockSpec(memory_space=pltpu.VMEM),
            pl.BlockSpec(memory_space=pltpu.VMEM),
            pl.BlockSpec(memory_space=pl.ANY),
        ],
        out_specs=pl.BlockSpec(memory_space=pltpu.VMEM),
        out_shape=jax.ShapeDtypeStruct((2, T), jnp.float32),
        scratch_shapes=[
            pltpu.SMEM((NW * MAXB,), jnp.int32),
            pltpu.VMEM((2, MBLK, D), jnp.float32),
            pltpu.VMEM((3, MBLK, W1.shape[1]), jnp.float32),
            pltpu.SemaphoreType.DMA((2,)),
        ],
    )(counts, W1, b1, W2, b2, xc)


def _run_scb(types, jv2, t16, nbatch):
    T = types.shape[0]
    TPW = T // NW
    WPB = NW // nbatch
    N = T // nbatch
    scb = pl.kernel(
        functools.partial(_scb_body, tpw=TPW, wpb=WPB, n=N),
        out_type=[
            jax.ShapeDtypeStruct((nbatch * E, N), jnp.float32),
            jax.ShapeDtypeStruct((nbatch * E, N), jnp.float32),
        ],
        mesh=_sc_mesh(),
        scratch_types=[
            pltpu.VMEM((TPW,), jnp.int32),
            pltpu.VMEM((TPW,), jnp.float32),
            pltpu.VMEM((TPW,), jnp.float32),
            pltpu.VMEM((L,), jnp.int32),
            pltpu.VMEM((E, TPW), jnp.float32),
            pltpu.VMEM((E, TPW), jnp.float32),
        ],
        compiler_params=pltpu.CompilerParams(needs_layout_passes=False),
    )
    return scb(types, jv2, t16)


def kernel(tokens, token_types, t, W1, b1, W2, b2):
    B, N, D = tokens.shape
    T = B * N

    x = tokens.reshape(T, D)
    types = token_types.reshape(T).astype(jnp.int32)
    t16 = jnp.zeros((L,), jnp.int32).at[:B].set(t.astype(jnp.int32))

    counts, xc = _run_sca(types, x)
    jv2 = _run_tc(counts, W1, b1, W2, b2, xc)
    disp_t, comb_t = _run_scb(types, jv2, t16, B)

    disp = disp_t.reshape(B, E, N).transpose(0, 2, 1)
    comb = comb_t.reshape(B, E, N).transpose(0, 2, 1)
    return disp, comb
